# Initial kernel scaffold; baseline (speedup 1.0000x reference)
import jax
import jax.numpy as jnp
from jax.experimental import pallas as pl


def kernel(text, audio, visual, Wpt, bpt, Wpa, bpa, Wpv, bpv, W1t, g1t, b1t, W2t, W1a, g1a, b1a, W2a, W1v, g1v, b1v, W2v, Ut, Vt, Ua, Va, Uv, Vv, Wf1, bf1, Wf2, bf2, Wf3, bf3):
    raise NotImplementedError("write your pallas kernel here")


# speedup vs baseline: 3.5982x; 3.5982x over previous
import jax
import jax.numpy as jnp
from jax import lax
from jax.experimental import pallas as pl

K = 10
N = 512
HID = 256
FPAD = 384
EPS = 1e-5
_HI = lax.Precision.HIGHEST


def _dgcnn_body(xin_ref, wp_ref, bp_ref, w1_ref, g1_ref, b1_ref, w2_ref, out_ref):
    x = xin_ref[0, 0]
    wp = wp_ref[0]
    x = jnp.maximum(
        lax.dot_general(x, wp, (((1,), (0,)), ((), ())),
                        preferred_element_type=jnp.float32, precision=_HI)
        + bp_ref[0], 0.0)

    xsq = jnp.sum(x * x, axis=1, keepdims=True)
    xa = jnp.concatenate([x, jnp.ones((N, 1), jnp.float32)], axis=1)
    xb = jnp.concatenate([2.0 * x, -xsq], axis=1)
    s = lax.dot_general(xa, xb, (((1,), (1,)), ((), ())),
                        preferred_element_type=jnp.float32, precision=_HI)

    gs = g1_ref[0] * (1.0 / jnp.sqrt(1.0 + EPS))
    w1a = w1_ref[0, :HID]
    w1b = w1_ref[0, HID:]
    p = lax.dot_general(x, (w1a - w1b) * gs, (((1,), (0,)), ((), ())),
                        preferred_element_type=jnp.float32, precision=_HI) + b1_ref[0]
    q = lax.dot_general(x, w1b * gs, (((1,), (0,)), ((), ())),
                        preferred_element_type=jnp.float32, precision=_HI)
    w2 = w2_ref[0]

    iota = lax.broadcasted_iota(jnp.int32, (N, N), 1)
    acc = None
    for _ in range(K):
        mx = jnp.max(s, axis=1, keepdims=True)
        cand = jnp.where(s == mx, iota, N)
        j = jnp.min(cand, axis=1, keepdims=True)
        sel = iota == j
        onehot = sel.astype(jnp.float32)
        nq = lax.dot_general(onehot, q, (((1,), (0,)), ((), ())),
                             preferred_element_type=jnp.float32, precision=_HI)
        h = jnp.maximum(p + nq, 0.0)
        hk = lax.dot_general(h, w2, (((1,), (0,)), ((), ())),
                             preferred_element_type=jnp.float32, precision=_HI)
        acc = hk if acc is None else jnp.maximum(acc, hk)
        s = jnp.where(sel, -jnp.inf, s)

    feat = jnp.mean(acc, axis=0, keepdims=True)
    out_ref[0, 0] = jnp.broadcast_to(feat, (8, HID))


def _fusion_body(feats_ref, us_ref, vs_ref, wf1_ref, bf1_ref, wf2_ref, bf2_ref,
                 wf3_ref, bf3_ref, out_ref):
    t = feats_ref[0]
    a = feats_ref[1]
    v = feats_ref[2]
    us = us_ref[...]
    vs = vs_ref[...]
    nu2 = jnp.sum(us * us, axis=1, keepdims=True)
    nv2 = jnp.sum(vs * vs, axis=1, keepdims=True)
    duv = jnp.sum(us * vs, axis=1, keepdims=True)
    nrm = jnp.sqrt(nu2) * jnp.sqrt(nv2)
    mu = 0.5 + 0.5 * duv / nrm
    aa = mu * nrm
    bb = (1.0 - mu) * nrm
    mean = aa / (aa + bb)
    var = aa * bb / ((aa + bb) ** 2 * (aa + bb + 1.0))
    score = mean / jnp.sqrt(var)
    e = jnp.exp(score - jnp.max(score, axis=0, keepdims=True))
    w = e / jnp.sum(e, axis=0, keepdims=True)

    fused = jnp.concatenate(
        [t * w[0:1, 0:1], a * w[1:2, 0:1], v * w[2:3, 0:1]], axis=1)
    h = jnp.maximum(
        lax.dot_general(fused, wf1_ref[...], (((1,), (0,)), ((), ())),
                        preferred_element_type=jnp.float32, precision=_HI)
        + bf1_ref[...], 0.0)
    h = jnp.maximum(
        lax.dot_general(h, wf2_ref[...], (((1,), (0,)), ((), ())),
                        preferred_element_type=jnp.float32, precision=_HI)
        + bf2_ref[...], 0.0)
    hp = jnp.concatenate([h, jnp.zeros((4, h.shape[1]), jnp.float32)], axis=0)
    out_ref[...] = lax.dot_general(hp, wf3_ref[...], (((1,), (0,)), ((), ())),
                                   preferred_element_type=jnp.float32,
                                   precision=_HI) + bf3_ref[...]


def _pad_feat(x, fin):
    xt = jnp.swapaxes(x, 1, 2)
    return jnp.pad(xt, ((0, 0), (0, 0), (0, FPAD - fin)))


def kernel(text, audio, visual, Wpt, bpt, Wpa, bpa, Wpv, bpv, W1t, g1t, b1t, W2t,
           W1a, g1a, b1a, W2a, W1v, g1v, b1v, W2v, Ut, Vt, Ua, Va, Uv, Vv,
           Wf1, bf1, Wf2, bf2, Wf3, bf3, interpret=False):
    B = text.shape[0]
    xin = jnp.stack([_pad_feat(text, Wpt.shape[0]),
                     _pad_feat(audio, Wpa.shape[0]),
                     _pad_feat(visual, Wpv.shape[0])])
    wp = jnp.stack([jnp.pad(Wpt, ((0, FPAD - Wpt.shape[0]), (0, 0))),
                    jnp.pad(Wpa, ((0, FPAD - Wpa.shape[0]), (0, 0))),
                    jnp.pad(Wpv, ((0, FPAD - Wpv.shape[0]), (0, 0)))])
    bp = jnp.stack([bpt, bpa, bpv])[:, None, :]
    w1 = jnp.stack([W1t, W1a, W1v])
    g1 = jnp.stack([g1t, g1a, g1v])[:, None, :]
    b1 = jnp.stack([b1t, b1a, b1v])[:, None, :]
    w2 = jnp.stack([W2t, W2a, W2v])

    feats8 = pl.pallas_call(
        _dgcnn_body,
        grid=(3, B),
        in_specs=[
            pl.BlockSpec((1, 1, N, FPAD), lambda m, b: (m, b, 0, 0)),
            pl.BlockSpec((1, FPAD, HID), lambda m, b: (m, 0, 0)),
            pl.BlockSpec((1, 1, HID), lambda m, b: (m, 0, 0)),
            pl.BlockSpec((1, 2 * HID, HID), lambda m, b: (m, 0, 0)),
            pl.BlockSpec((1, 1, HID), lambda m, b: (m, 0, 0)),
            pl.BlockSpec((1, 1, HID), lambda m, b: (m, 0, 0)),
            pl.BlockSpec((1, HID, HID), lambda m, b: (m, 0, 0)),
        ],
        out_specs=pl.BlockSpec((1, 1, 8, HID), lambda m, b: (m, b, 0, 0)),
        out_shape=jax.ShapeDtypeStruct((3, B, 8, HID), jnp.float32),
        interpret=interpret,
    )(xin, wp, bp, w1, g1, b1, w2)
    feats = feats8[:, :, 0, :]

    us = jnp.stack([Ut, Ua, Uv])
    vs = jnp.stack([Vt, Va, Vv])
    wf3p = jnp.pad(Wf3, ((0, 0), (0, 127)))
    bf3p = jnp.pad(bf3[None, :], ((0, 0), (0, 127)))

    outp = pl.pallas_call(
        _fusion_body,
        out_shape=jax.ShapeDtypeStruct((8, 128), jnp.float32),
        interpret=interpret,
    )(feats, us, vs, Wf1, bf1[None, :], Wf2, bf2[None, :], wf3p, bf3p)
    return outp[:B, :1]


# speedup vs baseline: 10.2779x; 2.8564x over previous
import jax
import jax.numpy as jnp
from jax import lax
from jax.experimental import pallas as pl

K = 10
N = 512
HID = 256
FPAD = 384
EPS = 1e-5
_HI = lax.Precision.HIGHEST


def _dgcnn_body(xin_ref, wp_ref, bp_ref, w1_ref, g1_ref, b1_ref, w2_ref, out_ref):
    x = xin_ref[0, 0]
    wp = wp_ref[0]
    x = jnp.maximum(
        lax.dot_general(x, wp, (((1,), (0,)), ((), ())),
                        preferred_element_type=jnp.float32, precision=_HI)
        + bp_ref[0], 0.0)

    xsq = jnp.sum(x * x, axis=1, keepdims=True)
    xa = jnp.concatenate([x, jnp.ones((N, 1), jnp.float32)], axis=1)
    xb = jnp.concatenate([2.0 * x, -xsq], axis=1)
    s = lax.dot_general(xa, xb, (((1,), (1,)), ((), ())),
                        preferred_element_type=jnp.float32, precision=_HI)

    gs = g1_ref[0] * (1.0 / jnp.sqrt(1.0 + EPS))
    w1a = w1_ref[0, :HID]
    w1b = w1_ref[0, HID:]
    p = lax.dot_general(x, (w1a - w1b) * gs, (((1,), (0,)), ((), ())),
                        preferred_element_type=jnp.float32) + b1_ref[0]
    q = lax.dot_general(x, w1b * gs, (((1,), (0,)), ((), ())),
                        preferred_element_type=jnp.float32)
    w2 = w2_ref[0]

    iota = lax.broadcasted_iota(jnp.int32, (N, N), 1)
    acc = None
    for _ in range(K):
        mx = jnp.max(s, axis=1, keepdims=True)
        cand = jnp.where(s == mx, iota, N)
        j = jnp.min(cand, axis=1, keepdims=True)
        sel = iota == j
        onehot = sel.astype(jnp.float32)
        nq = lax.dot_general(onehot, q, (((1,), (0,)), ((), ())),
                             preferred_element_type=jnp.float32)
        h = jnp.maximum(p + nq, 0.0)
        hk = lax.dot_general(h, w2, (((1,), (0,)), ((), ())),
                             preferred_element_type=jnp.float32)
        acc = hk if acc is None else jnp.maximum(acc, hk)
        s = jnp.where(sel, -jnp.inf, s)

    feat = jnp.mean(acc, axis=0, keepdims=True)
    out_ref[0, 0] = jnp.broadcast_to(feat, (8, HID))


def _fusion_body(feats_ref, us_ref, vs_ref, wf1_ref, bf1_ref, wf2_ref, bf2_ref,
                 wf3_ref, bf3_ref, out_ref):
    t = feats_ref[0]
    a = feats_ref[1]
    v = feats_ref[2]
    us = us_ref[...]
    vs = vs_ref[...]
    nu2 = jnp.sum(us * us, axis=1, keepdims=True)
    nv2 = jnp.sum(vs * vs, axis=1, keepdims=True)
    duv = jnp.sum(us * vs, axis=1, keepdims=True)
    nrm = jnp.sqrt(nu2) * jnp.sqrt(nv2)
    mu = 0.5 + 0.5 * duv / nrm
    aa = mu * nrm
    bb = (1.0 - mu) * nrm
    mean = aa / (aa + bb)
    var = aa * bb / ((aa + bb) ** 2 * (aa + bb + 1.0))
    score = mean / jnp.sqrt(var)
    e = jnp.exp(score - jnp.max(score, axis=0, keepdims=True))
    w = e / jnp.sum(e, axis=0, keepdims=True)

    fused = jnp.concatenate(
        [t * w[0:1, 0:1], a * w[1:2, 0:1], v * w[2:3, 0:1]], axis=1)
    h = jnp.maximum(
        lax.dot_general(fused, wf1_ref[...], (((1,), (0,)), ((), ())),
                        preferred_element_type=jnp.float32, precision=_HI)
        + bf1_ref[...], 0.0)
    h = jnp.maximum(
        lax.dot_general(h, wf2_ref[...], (((1,), (0,)), ((), ())),
                        preferred_element_type=jnp.float32, precision=_HI)
        + bf2_ref[...], 0.0)
    hp = jnp.concatenate([h, jnp.zeros((4, h.shape[1]), jnp.float32)], axis=0)
    out_ref[...] = lax.dot_general(hp, wf3_ref[...], (((1,), (0,)), ((), ())),
                                   preferred_element_type=jnp.float32,
                                   precision=_HI) + bf3_ref[...]


def _pad_feat(x, fin):
    xt = jnp.swapaxes(x, 1, 2)
    return jnp.pad(xt, ((0, 0), (0, 0), (0, FPAD - fin)))


def kernel(text, audio, visual, Wpt, bpt, Wpa, bpa, Wpv, bpv, W1t, g1t, b1t, W2t,
           W1a, g1a, b1a, W2a, W1v, g1v, b1v, W2v, Ut, Vt, Ua, Va, Uv, Vv,
           Wf1, bf1, Wf2, bf2, Wf3, bf3, interpret=False):
    B = text.shape[0]
    xin = jnp.stack([_pad_feat(text, Wpt.shape[0]),
                     _pad_feat(audio, Wpa.shape[0]),
                     _pad_feat(visual, Wpv.shape[0])])
    wp = jnp.stack([jnp.pad(Wpt, ((0, FPAD - Wpt.shape[0]), (0, 0))),
                    jnp.pad(Wpa, ((0, FPAD - Wpa.shape[0]), (0, 0))),
                    jnp.pad(Wpv, ((0, FPAD - Wpv.shape[0]), (0, 0)))])
    bp = jnp.stack([bpt, bpa, bpv])[:, None, :]
    w1 = jnp.stack([W1t, W1a, W1v])
    g1 = jnp.stack([g1t, g1a, g1v])[:, None, :]
    b1 = jnp.stack([b1t, b1a, b1v])[:, None, :]
    w2 = jnp.stack([W2t, W2a, W2v])

    feats8 = pl.pallas_call(
        _dgcnn_body,
        grid=(3, B),
        in_specs=[
            pl.BlockSpec((1, 1, N, FPAD), lambda m, b: (m, b, 0, 0)),
            pl.BlockSpec((1, FPAD, HID), lambda m, b: (m, 0, 0)),
            pl.BlockSpec((1, 1, HID), lambda m, b: (m, 0, 0)),
            pl.BlockSpec((1, 2 * HID, HID), lambda m, b: (m, 0, 0)),
            pl.BlockSpec((1, 1, HID), lambda m, b: (m, 0, 0)),
            pl.BlockSpec((1, 1, HID), lambda m, b: (m, 0, 0)),
            pl.BlockSpec((1, HID, HID), lambda m, b: (m, 0, 0)),
        ],
        out_specs=pl.BlockSpec((1, 1, 8, HID), lambda m, b: (m, b, 0, 0)),
        out_shape=jax.ShapeDtypeStruct((3, B, 8, HID), jnp.float32),
        interpret=interpret,
    )(xin, wp, bp, w1, g1, b1, w2)
    feats = feats8[:, :, 0, :]

    us = jnp.stack([Ut, Ua, Uv])
    vs = jnp.stack([Vt, Va, Vv])
    wf3p = jnp.pad(Wf3, ((0, 0), (0, 127)))
    bf3p = jnp.pad(bf3[None, :], ((0, 0), (0, 127)))

    outp = pl.pallas_call(
        _fusion_body,
        out_shape=jax.ShapeDtypeStruct((8, 128), jnp.float32),
        interpret=interpret,
    )(feats, us, vs, Wf1, bf1[None, :], Wf2, bf2[None, :], wf3p, bf3p)
    return outp[:B, :1]


# speedup vs baseline: 10.9886x; 1.0692x over previous
import jax
import jax.numpy as jnp
from jax import lax
from jax.experimental import pallas as pl

K = 10
N = 512
HID = 256
FPAD = 384
EPS = 1e-5
_HI = lax.Precision.HIGHEST


def _dgcnn_body(xin_ref, wp_ref, bp_ref, w1_ref, g1_ref, b1_ref, w2_ref, out_ref):
    x = xin_ref[0, 0]
    wp = wp_ref[0]
    x = jnp.maximum(
        lax.dot_general(x, wp, (((1,), (0,)), ((), ())),
                        preferred_element_type=jnp.float32, precision=_HI)
        + bp_ref[0], 0.0)

    xsq = jnp.sum(x * x, axis=1, keepdims=True)
    xa = jnp.concatenate([x, jnp.ones((N, 1), jnp.float32)], axis=1)
    xb = jnp.concatenate([2.0 * x, -xsq], axis=1)
    s = lax.dot_general(xa, xb, (((1,), (1,)), ((), ())),
                        preferred_element_type=jnp.float32, precision=_HI)

    gs = g1_ref[0] * (1.0 / jnp.sqrt(1.0 + EPS))
    w1a = w1_ref[0, :HID]
    w1b = w1_ref[0, HID:]
    p = lax.dot_general(x, (w1a - w1b) * gs, (((1,), (0,)), ((), ())),
                        preferred_element_type=jnp.float32) + b1_ref[0]
    q = lax.dot_general(x, w1b * gs, (((1,), (0,)), ((), ())),
                        preferred_element_type=jnp.float32)
    w2 = w2_ref[0]

    iota = lax.broadcasted_iota(jnp.int32, (N, N), 1)
    h0 = jnp.maximum(p + q, 0.0)
    acc = lax.dot_general(h0, w2, (((1,), (0,)), ((), ())),
                          preferred_element_type=jnp.float32)
    s = jnp.where(iota == lax.broadcasted_iota(jnp.int32, (N, N), 0),
                  -jnp.inf, s)
    for _ in range(K - 1):
        mx = jnp.max(s, axis=1, keepdims=True)
        cand = jnp.where(s == mx, iota, N)
        j = jnp.min(cand, axis=1, keepdims=True)
        sel = iota == j
        onehot = sel.astype(jnp.float32)
        nq = lax.dot_general(onehot, q, (((1,), (0,)), ((), ())),
                             preferred_element_type=jnp.float32)
        h = jnp.maximum(p + nq, 0.0)
        hk = lax.dot_general(h, w2, (((1,), (0,)), ((), ())),
                             preferred_element_type=jnp.float32)
        acc = jnp.maximum(acc, hk)
        s = jnp.where(sel, -jnp.inf, s)

    feat = jnp.mean(acc, axis=0, keepdims=True)
    out_ref[0, 0] = jnp.broadcast_to(feat, (8, HID))


def _fusion_body(feats_ref, us_ref, vs_ref, wf1_ref, bf1_ref, wf2_ref, bf2_ref,
                 wf3_ref, bf3_ref, out_ref):
    t = feats_ref[0]
    a = feats_ref[1]
    v = feats_ref[2]
    us = us_ref[...]
    vs = vs_ref[...]
    nu2 = jnp.sum(us * us, axis=1, keepdims=True)
    nv2 = jnp.sum(vs * vs, axis=1, keepdims=True)
    duv = jnp.sum(us * vs, axis=1, keepdims=True)
    nrm = jnp.sqrt(nu2) * jnp.sqrt(nv2)
    mu = 0.5 + 0.5 * duv / nrm
    aa = mu * nrm
    bb = (1.0 - mu) * nrm
    mean = aa / (aa + bb)
    var = aa * bb / ((aa + bb) ** 2 * (aa + bb + 1.0))
    score = mean / jnp.sqrt(var)
    e = jnp.exp(score - jnp.max(score, axis=0, keepdims=True))
    w = e / jnp.sum(e, axis=0, keepdims=True)

    fused = jnp.concatenate(
        [t * w[0:1, 0:1], a * w[1:2, 0:1], v * w[2:3, 0:1]], axis=1)
    h = jnp.maximum(
        lax.dot_general(fused, wf1_ref[...], (((1,), (0,)), ((), ())),
                        preferred_element_type=jnp.float32, precision=_HI)
        + bf1_ref[...], 0.0)
    h = jnp.maximum(
        lax.dot_general(h, wf2_ref[...], (((1,), (0,)), ((), ())),
                        preferred_element_type=jnp.float32, precision=_HI)
        + bf2_ref[...], 0.0)
    hp = jnp.concatenate([h, jnp.zeros((4, h.shape[1]), jnp.float32)], axis=0)
    out_ref[...] = lax.dot_general(hp, wf3_ref[...], (((1,), (0,)), ((), ())),
                                   preferred_element_type=jnp.float32,
                                   precision=_HI) + bf3_ref[...]


def _pad_feat(x, fin):
    xt = jnp.swapaxes(x, 1, 2)
    return jnp.pad(xt, ((0, 0), (0, 0), (0, FPAD - fin)))


def kernel(text, audio, visual, Wpt, bpt, Wpa, bpa, Wpv, bpv, W1t, g1t, b1t, W2t,
           W1a, g1a, b1a, W2a, W1v, g1v, b1v, W2v, Ut, Vt, Ua, Va, Uv, Vv,
           Wf1, bf1, Wf2, bf2, Wf3, bf3, interpret=False):
    B = text.shape[0]
    xin = jnp.stack([_pad_feat(text, Wpt.shape[0]),
                     _pad_feat(audio, Wpa.shape[0]),
                     _pad_feat(visual, Wpv.shape[0])])
    wp = jnp.stack([jnp.pad(Wpt, ((0, FPAD - Wpt.shape[0]), (0, 0))),
                    jnp.pad(Wpa, ((0, FPAD - Wpa.shape[0]), (0, 0))),
                    jnp.pad(Wpv, ((0, FPAD - Wpv.shape[0]), (0, 0)))])
    bp = jnp.stack([bpt, bpa, bpv])[:, None, :]
    w1 = jnp.stack([W1t, W1a, W1v])
    g1 = jnp.stack([g1t, g1a, g1v])[:, None, :]
    b1 = jnp.stack([b1t, b1a, b1v])[:, None, :]
    w2 = jnp.stack([W2t, W2a, W2v])

    feats8 = pl.pallas_call(
        _dgcnn_body,
        grid=(3, B),
        in_specs=[
            pl.BlockSpec((1, 1, N, FPAD), lambda m, b: (m, b, 0, 0)),
            pl.BlockSpec((1, FPAD, HID), lambda m, b: (m, 0, 0)),
            pl.BlockSpec((1, 1, HID), lambda m, b: (m, 0, 0)),
            pl.BlockSpec((1, 2 * HID, HID), lambda m, b: (m, 0, 0)),
            pl.BlockSpec((1, 1, HID), lambda m, b: (m, 0, 0)),
            pl.BlockSpec((1, 1, HID), lambda m, b: (m, 0, 0)),
            pl.BlockSpec((1, HID, HID), lambda m, b: (m, 0, 0)),
        ],
        out_specs=pl.BlockSpec((1, 1, 8, HID), lambda m, b: (m, b, 0, 0)),
        out_shape=jax.ShapeDtypeStruct((3, B, 8, HID), jnp.float32),
        interpret=interpret,
    )(xin, wp, bp, w1, g1, b1, w2)
    feats = feats8[:, :, 0, :]

    us = jnp.stack([Ut, Ua, Uv])
    vs = jnp.stack([Vt, Va, Vv])
    wf3p = jnp.pad(Wf3, ((0, 0), (0, 127)))
    bf3p = jnp.pad(bf3[None, :], ((0, 0), (0, 127)))

    outp = pl.pallas_call(
        _fusion_body,
        out_shape=jax.ShapeDtypeStruct((8, 128), jnp.float32),
        interpret=interpret,
    )(feats, us, vs, Wf1, bf1[None, :], Wf2, bf2[None, :], wf3p, bf3p)
    return outp[:B, :1]


# speedup vs baseline: 12.5059x; 1.1381x over previous
import jax
import jax.numpy as jnp
from jax import lax
from jax.experimental import pallas as pl

K = 10
N = 512
HID = 256
FPAD = 384
EPS = 1e-5
_HI = lax.Precision.HIGHEST


def _split_bf16(a):
    hi = a.astype(jnp.bfloat16)
    lo = (a - hi.astype(jnp.float32)).astype(jnp.bfloat16)
    return hi, lo


def _dot3(a, b, dims):
    a_hi, a_lo = _split_bf16(a)
    b_hi, b_lo = _split_bf16(b)
    dn = (dims, ((), ()))
    out = lax.dot_general(a_hi, b_hi, dn, preferred_element_type=jnp.float32)
    out += lax.dot_general(a_hi, b_lo, dn, preferred_element_type=jnp.float32)
    out += lax.dot_general(a_lo, b_hi, dn, preferred_element_type=jnp.float32)
    return out


def _dgcnn_body(xin_ref, wp_ref, bp_ref, w1_ref, g1_ref, b1_ref, w2_ref, out_ref):
    x = xin_ref[0, 0]
    wp = wp_ref[0]
    x = jnp.maximum(
        lax.dot_general(x, wp, (((1,), (0,)), ((), ())),
                        preferred_element_type=jnp.float32)
        + bp_ref[0], 0.0)

    xsq = jnp.sum(x * x, axis=1, keepdims=True)
    xa = jnp.concatenate([x, jnp.ones((N, 1), jnp.float32)], axis=1)
    xb = jnp.concatenate([2.0 * x, -xsq], axis=1)
    s = lax.dot_general(xa, xb, (((1,), (1,)), ((), ())),
                        preferred_element_type=jnp.float32, precision=_HI)

    gs = g1_ref[0] * (1.0 / jnp.sqrt(1.0 + EPS))
    w1a = w1_ref[0, :HID]
    w1b = w1_ref[0, HID:]
    p = lax.dot_general(x, (w1a - w1b) * gs, (((1,), (0,)), ((), ())),
                        preferred_element_type=jnp.float32) + b1_ref[0]
    q = lax.dot_general(x, w1b * gs, (((1,), (0,)), ((), ())),
                        preferred_element_type=jnp.float32)
    w2 = w2_ref[0]

    iota = lax.broadcasted_iota(jnp.int32, (N, N), 1)
    h0 = jnp.maximum(p + q, 0.0)
    acc = lax.dot_general(h0, w2, (((1,), (0,)), ((), ())),
                          preferred_element_type=jnp.float32)
    s = jnp.where(iota == lax.broadcasted_iota(jnp.int32, (N, N), 0),
                  -jnp.inf, s)
    for _ in range(K - 1):
        mx = jnp.max(s, axis=1, keepdims=True)
        cand = jnp.where(s == mx, iota, N)
        j = jnp.min(cand, axis=1, keepdims=True)
        sel = iota == j
        onehot = sel.astype(jnp.float32)
        nq = lax.dot_general(onehot, q, (((1,), (0,)), ((), ())),
                             preferred_element_type=jnp.float32)
        h = jnp.maximum(p + nq, 0.0)
        hk = lax.dot_general(h, w2, (((1,), (0,)), ((), ())),
                             preferred_element_type=jnp.float32)
        acc = jnp.maximum(acc, hk)
        s = jnp.where(sel, -jnp.inf, s)

    feat = jnp.mean(acc, axis=0, keepdims=True)
    out_ref[0, 0] = jnp.broadcast_to(feat, (8, HID))


def _fusion_body(feats_ref, us_ref, vs_ref, wf1_ref, bf1_ref, wf2_ref, bf2_ref,
                 wf3_ref, bf3_ref, out_ref):
    t = feats_ref[0]
    a = feats_ref[1]
    v = feats_ref[2]
    us = us_ref[...]
    vs = vs_ref[...]
    nu2 = jnp.sum(us * us, axis=1, keepdims=True)
    nv2 = jnp.sum(vs * vs, axis=1, keepdims=True)
    duv = jnp.sum(us * vs, axis=1, keepdims=True)
    nrm = jnp.sqrt(nu2) * jnp.sqrt(nv2)
    mu = 0.5 + 0.5 * duv / nrm
    aa = mu * nrm
    bb = (1.0 - mu) * nrm
    mean = aa / (aa + bb)
    var = aa * bb / ((aa + bb) ** 2 * (aa + bb + 1.0))
    score = mean / jnp.sqrt(var)
    e = jnp.exp(score - jnp.max(score, axis=0, keepdims=True))
    w = e / jnp.sum(e, axis=0, keepdims=True)

    fused = jnp.concatenate(
        [t * w[0:1, 0:1], a * w[1:2, 0:1], v * w[2:3, 0:1]], axis=1)
    h = jnp.maximum(
        lax.dot_general(fused, wf1_ref[...], (((1,), (0,)), ((), ())),
                        preferred_element_type=jnp.float32)
        + bf1_ref[...], 0.0)
    h = jnp.maximum(
        lax.dot_general(h, wf2_ref[...], (((1,), (0,)), ((), ())),
                        preferred_element_type=jnp.float32)
        + bf2_ref[...], 0.0)
    hp = jnp.concatenate([h, jnp.zeros((4, h.shape[1]), jnp.float32)], axis=0)
    out_ref[...] = lax.dot_general(hp, wf3_ref[...], (((1,), (0,)), ((), ())),
                                   preferred_element_type=jnp.float32
                                   ) + bf3_ref[...]


def _pad_feat(x, fin):
    xt = jnp.swapaxes(x, 1, 2)
    return jnp.pad(xt, ((0, 0), (0, 0), (0, FPAD - fin)))


def kernel(text, audio, visual, Wpt, bpt, Wpa, bpa, Wpv, bpv, W1t, g1t, b1t, W2t,
           W1a, g1a, b1a, W2a, W1v, g1v, b1v, W2v, Ut, Vt, Ua, Va, Uv, Vv,
           Wf1, bf1, Wf2, bf2, Wf3, bf3, interpret=False):
    B = text.shape[0]
    xin = jnp.stack([_pad_feat(text, Wpt.shape[0]),
                     _pad_feat(audio, Wpa.shape[0]),
                     _pad_feat(visual, Wpv.shape[0])])
    wp = jnp.stack([jnp.pad(Wpt, ((0, FPAD - Wpt.shape[0]), (0, 0))),
                    jnp.pad(Wpa, ((0, FPAD - Wpa.shape[0]), (0, 0))),
                    jnp.pad(Wpv, ((0, FPAD - Wpv.shape[0]), (0, 0)))])
    bp = jnp.stack([bpt, bpa, bpv])[:, None, :]
    w1 = jnp.stack([W1t, W1a, W1v])
    g1 = jnp.stack([g1t, g1a, g1v])[:, None, :]
    b1 = jnp.stack([b1t, b1a, b1v])[:, None, :]
    w2 = jnp.stack([W2t, W2a, W2v])

    feats8 = pl.pallas_call(
        _dgcnn_body,
        grid=(3, B),
        in_specs=[
            pl.BlockSpec((1, 1, N, FPAD), lambda m, b: (m, b, 0, 0)),
            pl.BlockSpec((1, FPAD, HID), lambda m, b: (m, 0, 0)),
            pl.BlockSpec((1, 1, HID), lambda m, b: (m, 0, 0)),
            pl.BlockSpec((1, 2 * HID, HID), lambda m, b: (m, 0, 0)),
            pl.BlockSpec((1, 1, HID), lambda m, b: (m, 0, 0)),
            pl.BlockSpec((1, 1, HID), lambda m, b: (m, 0, 0)),
            pl.BlockSpec((1, HID, HID), lambda m, b: (m, 0, 0)),
        ],
        out_specs=pl.BlockSpec((1, 1, 8, HID), lambda m, b: (m, b, 0, 0)),
        out_shape=jax.ShapeDtypeStruct((3, B, 8, HID), jnp.float32),
        interpret=interpret,
    )(xin, wp, bp, w1, g1, b1, w2)
    feats = feats8[:, :, 0, :]

    us = jnp.stack([Ut, Ua, Uv])
    vs = jnp.stack([Vt, Va, Vv])
    wf3p = jnp.pad(Wf3, ((0, 0), (0, 127)))
    bf3p = jnp.pad(bf3[None, :], ((0, 0), (0, 127)))

    outp = pl.pallas_call(
        _fusion_body,
        out_shape=jax.ShapeDtypeStruct((8, 128), jnp.float32),
        interpret=interpret,
    )(feats, us, vs, Wf1, bf1[None, :], Wf2, bf2[None, :], wf3p, bf3p)
    return outp[:B, :1]


# speedup vs baseline: 13.8884x; 1.1105x over previous
import jax
import jax.numpy as jnp
from jax import lax
from jax.experimental import pallas as pl

K = 10
N = 512
HID = 256
FPAD = 384
EPS = 1e-5
_HI = lax.Precision.HIGHEST


def _split_bf16(a):
    hi = a.astype(jnp.bfloat16)
    lo = (a - hi.astype(jnp.float32)).astype(jnp.bfloat16)
    return hi, lo


def _dot3(a, b, dims):
    a_hi, a_lo = _split_bf16(a)
    b_hi, b_lo = _split_bf16(b)
    dn = (dims, ((), ()))
    out = lax.dot_general(a_hi, b_hi, dn, preferred_element_type=jnp.float32)
    out += lax.dot_general(a_hi, b_lo, dn, preferred_element_type=jnp.float32)
    out += lax.dot_general(a_lo, b_hi, dn, preferred_element_type=jnp.float32)
    return out


def _dgcnn_body(xin_ref, wp_ref, bp_ref, w1_ref, g1_ref, b1_ref, w2_ref, out_ref):
    x = xin_ref[0, 0]
    wp = wp_ref[0]
    x = jnp.maximum(
        lax.dot_general(x, wp, (((1,), (0,)), ((), ())),
                        preferred_element_type=jnp.float32)
        + bp_ref[0], 0.0)

    xsq = jnp.sum(x * x, axis=1, keepdims=True)
    xa = jnp.concatenate([x, jnp.ones((N, 1), jnp.float32)], axis=1)
    xb = jnp.concatenate([2.0 * x, -xsq], axis=1)
    s = _dot3(xa, xb, ((1,), (1,)))

    gs = g1_ref[0] * (1.0 / jnp.sqrt(1.0 + EPS))
    w1a = w1_ref[0, :HID]
    w1b = w1_ref[0, HID:]
    p = lax.dot_general(x, (w1a - w1b) * gs, (((1,), (0,)), ((), ())),
                        preferred_element_type=jnp.float32) + b1_ref[0]
    q = lax.dot_general(x, w1b * gs, (((1,), (0,)), ((), ())),
                        preferred_element_type=jnp.float32)
    w2 = w2_ref[0]

    iota = lax.broadcasted_iota(jnp.int32, (N, N), 1)
    h0 = jnp.maximum(p + q, 0.0)
    acc = lax.dot_general(h0, w2, (((1,), (0,)), ((), ())),
                          preferred_element_type=jnp.float32)
    s = jnp.where(iota == lax.broadcasted_iota(jnp.int32, (N, N), 0),
                  -jnp.inf, s)
    for _ in range(K - 1):
        mx = jnp.max(s, axis=1, keepdims=True)
        cand = jnp.where(s == mx, iota, N)
        j = jnp.min(cand, axis=1, keepdims=True)
        sel = iota == j
        onehot = sel.astype(jnp.float32)
        nq = lax.dot_general(onehot, q, (((1,), (0,)), ((), ())),
                             preferred_element_type=jnp.float32)
        h = jnp.maximum(p + nq, 0.0)
        hk = lax.dot_general(h, w2, (((1,), (0,)), ((), ())),
                             preferred_element_type=jnp.float32)
        acc = jnp.maximum(acc, hk)
        s = jnp.where(sel, -jnp.inf, s)

    feat = jnp.mean(acc, axis=0, keepdims=True)
    out_ref[0, 0] = jnp.broadcast_to(feat, (8, HID))


def _fusion_body(feats_ref, us_ref, vs_ref, wf1_ref, bf1_ref, wf2_ref, bf2_ref,
                 wf3_ref, bf3_ref, out_ref):
    t = feats_ref[0]
    a = feats_ref[1]
    v = feats_ref[2]
    us = us_ref[...]
    vs = vs_ref[...]
    nu2 = jnp.sum(us * us, axis=1, keepdims=True)
    nv2 = jnp.sum(vs * vs, axis=1, keepdims=True)
    duv = jnp.sum(us * vs, axis=1, keepdims=True)
    nrm = jnp.sqrt(nu2) * jnp.sqrt(nv2)
    mu = 0.5 + 0.5 * duv / nrm
    aa = mu * nrm
    bb = (1.0 - mu) * nrm
    mean = aa / (aa + bb)
    var = aa * bb / ((aa + bb) ** 2 * (aa + bb + 1.0))
    score = mean / jnp.sqrt(var)
    e = jnp.exp(score - jnp.max(score, axis=0, keepdims=True))
    w = e / jnp.sum(e, axis=0, keepdims=True)

    fused = jnp.concatenate(
        [t * w[0:1, 0:1], a * w[1:2, 0:1], v * w[2:3, 0:1]], axis=1)
    h = jnp.maximum(
        lax.dot_general(fused, wf1_ref[...], (((1,), (0,)), ((), ())),
                        preferred_element_type=jnp.float32)
        + bf1_ref[...], 0.0)
    h = jnp.maximum(
        lax.dot_general(h, wf2_ref[...], (((1,), (0,)), ((), ())),
                        preferred_element_type=jnp.float32)
        + bf2_ref[...], 0.0)
    hp = jnp.concatenate([h, jnp.zeros((4, h.shape[1]), jnp.float32)], axis=0)
    out_ref[...] = lax.dot_general(hp, wf3_ref[...], (((1,), (0,)), ((), ())),
                                   preferred_element_type=jnp.float32
                                   ) + bf3_ref[...]


def _pad_feat(x, fin):
    xt = jnp.swapaxes(x, 1, 2)
    return jnp.pad(xt, ((0, 0), (0, 0), (0, FPAD - fin)))


def kernel(text, audio, visual, Wpt, bpt, Wpa, bpa, Wpv, bpv, W1t, g1t, b1t, W2t,
           W1a, g1a, b1a, W2a, W1v, g1v, b1v, W2v, Ut, Vt, Ua, Va, Uv, Vv,
           Wf1, bf1, Wf2, bf2, Wf3, bf3, interpret=False):
    B = text.shape[0]
    xin = jnp.stack([_pad_feat(text, Wpt.shape[0]),
                     _pad_feat(audio, Wpa.shape[0]),
                     _pad_feat(visual, Wpv.shape[0])])
    wp = jnp.stack([jnp.pad(Wpt, ((0, FPAD - Wpt.shape[0]), (0, 0))),
                    jnp.pad(Wpa, ((0, FPAD - Wpa.shape[0]), (0, 0))),
                    jnp.pad(Wpv, ((0, FPAD - Wpv.shape[0]), (0, 0)))])
    bp = jnp.stack([bpt, bpa, bpv])[:, None, :]
    w1 = jnp.stack([W1t, W1a, W1v])
    g1 = jnp.stack([g1t, g1a, g1v])[:, None, :]
    b1 = jnp.stack([b1t, b1a, b1v])[:, None, :]
    w2 = jnp.stack([W2t, W2a, W2v])

    feats8 = pl.pallas_call(
        _dgcnn_body,
        grid=(3, B),
        in_specs=[
            pl.BlockSpec((1, 1, N, FPAD), lambda m, b: (m, b, 0, 0)),
            pl.BlockSpec((1, FPAD, HID), lambda m, b: (m, 0, 0)),
            pl.BlockSpec((1, 1, HID), lambda m, b: (m, 0, 0)),
            pl.BlockSpec((1, 2 * HID, HID), lambda m, b: (m, 0, 0)),
            pl.BlockSpec((1, 1, HID), lambda m, b: (m, 0, 0)),
            pl.BlockSpec((1, 1, HID), lambda m, b: (m, 0, 0)),
            pl.BlockSpec((1, HID, HID), lambda m, b: (m, 0, 0)),
        ],
        out_specs=pl.BlockSpec((1, 1, 8, HID), lambda m, b: (m, b, 0, 0)),
        out_shape=jax.ShapeDtypeStruct((3, B, 8, HID), jnp.float32),
        interpret=interpret,
    )(xin, wp, bp, w1, g1, b1, w2)
    feats = feats8[:, :, 0, :]

    us = jnp.stack([Ut, Ua, Uv])
    vs = jnp.stack([Vt, Va, Vv])
    wf3p = jnp.pad(Wf3, ((0, 0), (0, 127)))
    bf3p = jnp.pad(bf3[None, :], ((0, 0), (0, 127)))

    outp = pl.pallas_call(
        _fusion_body,
        out_shape=jax.ShapeDtypeStruct((8, 128), jnp.float32),
        interpret=interpret,
    )(feats, us, vs, Wf1, bf1[None, :], Wf2, bf2[None, :], wf3p, bf3p)
    return outp[:B, :1]


# speedup vs baseline: 15.4830x; 1.1148x over previous
import jax
import jax.numpy as jnp
from jax import lax
from jax.experimental import pallas as pl

K = 10
N = 512
HID = 256
EPS = 1e-5


def _split_bf16(a):
    hi = a.astype(jnp.bfloat16)
    lo = (a - hi.astype(jnp.float32)).astype(jnp.bfloat16)
    return hi, lo


def _dot3(a, b, dims):
    a_hi, a_lo = _split_bf16(a)
    b_hi, b_lo = _split_bf16(b)
    dn = (dims, ((), ()))
    out = lax.dot_general(a_hi, b_hi, dn, preferred_element_type=jnp.float32)
    out += lax.dot_general(a_hi, b_lo, dn, preferred_element_type=jnp.float32)
    out += lax.dot_general(a_lo, b_hi, dn, preferred_element_type=jnp.float32)
    return out


def _dgcnn_body(xin_ref, wp_ref, bp_ref, w1_ref, g1_ref, b1_ref, w2_ref, out_ref):
    xin = xin_ref[0]
    wp = wp_ref[...]
    x = jnp.maximum(
        lax.dot_general(xin, wp, (((0,), (0,)), ((), ())),
                        preferred_element_type=jnp.float32)
        + bp_ref[...], 0.0)

    xsq = jnp.sum(x * x, axis=1, keepdims=True)
    xa = jnp.concatenate([x, jnp.ones((N, 1), jnp.float32)], axis=1)
    xb = jnp.concatenate([2.0 * x, -xsq], axis=1)
    s = _dot3(xa, xb, ((1,), (1,)))

    gs = g1_ref[...] * (1.0 / jnp.sqrt(1.0 + EPS))
    w1a = w1_ref[:HID]
    w1b = w1_ref[HID:]
    p = lax.dot_general(x, (w1a - w1b) * gs, (((1,), (0,)), ((), ())),
                        preferred_element_type=jnp.float32) + b1_ref[...]
    q = lax.dot_general(x, w1b * gs, (((1,), (0,)), ((), ())),
                        preferred_element_type=jnp.float32)
    w2 = w2_ref[...]

    iota = lax.broadcasted_iota(jnp.int32, (N, N), 1)
    h0 = jnp.maximum(p + q, 0.0)
    acc = lax.dot_general(h0, w2, (((1,), (0,)), ((), ())),
                          preferred_element_type=jnp.float32)
    s = jnp.where(iota == lax.broadcasted_iota(jnp.int32, (N, N), 0),
                  -jnp.inf, s)
    for _ in range(K - 1):
        mx = jnp.max(s, axis=1, keepdims=True)
        cand = jnp.where(s == mx, iota, N)
        j = jnp.min(cand, axis=1, keepdims=True)
        sel = iota == j
        onehot = sel.astype(jnp.float32)
        nq = lax.dot_general(onehot, q, (((1,), (0,)), ((), ())),
                             preferred_element_type=jnp.float32)
        h = jnp.maximum(p + nq, 0.0)
        hk = lax.dot_general(h, w2, (((1,), (0,)), ((), ())),
                             preferred_element_type=jnp.float32)
        acc = jnp.maximum(acc, hk)
        s = jnp.where(sel, -jnp.inf, s)

    feat = jnp.mean(acc, axis=0, keepdims=True)
    out_ref[0] = jnp.broadcast_to(feat, (8, HID))


def _dgcnn(xraw, wp, bp, w1, g1, b1, w2):
    B, fin, _ = xraw.shape
    feats8 = pl.pallas_call(
        _dgcnn_body,
        grid=(B,),
        in_specs=[
            pl.BlockSpec((1, fin, N), lambda b: (b, 0, 0)),
            pl.BlockSpec((fin, HID), lambda b: (0, 0)),
            pl.BlockSpec((1, HID), lambda b: (0, 0)),
            pl.BlockSpec((2 * HID, HID), lambda b: (0, 0)),
            pl.BlockSpec((1, HID), lambda b: (0, 0)),
            pl.BlockSpec((1, HID), lambda b: (0, 0)),
            pl.BlockSpec((HID, HID), lambda b: (0, 0)),
        ],
        out_specs=pl.BlockSpec((1, 8, HID), lambda b: (b, 0, 0)),
        out_shape=jax.ShapeDtypeStruct((B, 8, HID), jnp.float32),
    )(xraw, wp, bp[None, :], w1, g1[None, :], b1[None, :], w2)
    return feats8[:, 0, :]


def _fusion_body(feats_ref, us_ref, vs_ref, wf1_ref, bf1_ref, wf2_ref, bf2_ref,
                 wf3_ref, bf3_ref, out_ref):
    t = feats_ref[0]
    a = feats_ref[1]
    v = feats_ref[2]
    us = us_ref[...]
    vs = vs_ref[...]
    nu2 = jnp.sum(us * us, axis=1, keepdims=True)
    nv2 = jnp.sum(vs * vs, axis=1, keepdims=True)
    duv = jnp.sum(us * vs, axis=1, keepdims=True)
    nrm = jnp.sqrt(nu2) * jnp.sqrt(nv2)
    mu = 0.5 + 0.5 * duv / nrm
    aa = mu * nrm
    bb = (1.0 - mu) * nrm
    mean = aa / (aa + bb)
    var = aa * bb / ((aa + bb) ** 2 * (aa + bb + 1.0))
    score = mean / jnp.sqrt(var)
    e = jnp.exp(score - jnp.max(score, axis=0, keepdims=True))
    w = e / jnp.sum(e, axis=0, keepdims=True)

    fused = jnp.concatenate(
        [t * w[0:1, 0:1], a * w[1:2, 0:1], v * w[2:3, 0:1]], axis=1)
    h = jnp.maximum(
        lax.dot_general(fused, wf1_ref[...], (((1,), (0,)), ((), ())),
                        preferred_element_type=jnp.float32)
        + bf1_ref[...], 0.0)
    h = jnp.maximum(
        lax.dot_general(h, wf2_ref[...], (((1,), (0,)), ((), ())),
                        preferred_element_type=jnp.float32)
        + bf2_ref[...], 0.0)
    hp = jnp.concatenate([h, jnp.zeros((4, h.shape[1]), jnp.float32)], axis=0)
    out_ref[...] = lax.dot_general(hp, wf3_ref[...], (((1,), (0,)), ((), ())),
                                   preferred_element_type=jnp.float32
                                   ) + bf3_ref[...]


def kernel(text, audio, visual, Wpt, bpt, Wpa, bpa, Wpv, bpv, W1t, g1t, b1t, W2t,
           W1a, g1a, b1a, W2a, W1v, g1v, b1v, W2v, Ut, Vt, Ua, Va, Uv, Vv,
           Wf1, bf1, Wf2, bf2, Wf3, bf3):
    B = text.shape[0]
    t = _dgcnn(text, Wpt, bpt, W1t, g1t, b1t, W2t)
    a = _dgcnn(audio, Wpa, bpa, W1a, g1a, b1a, W2a)
    v = _dgcnn(visual, Wpv, bpv, W1v, g1v, b1v, W2v)
    feats = jnp.stack([t, a, v])

    us = jnp.stack([Ut, Ua, Uv])
    vs = jnp.stack([Vt, Va, Vv])
    wf3p = jnp.pad(Wf3, ((0, 0), (0, 127)))
    bf3p = jnp.pad(bf3[None, :], ((0, 0), (0, 127)))

    outp = pl.pallas_call(
        _fusion_body,
        out_shape=jax.ShapeDtypeStruct((8, 128), jnp.float32),
    )(feats, us, vs, Wf1, bf1[None, :], Wf2, bf2[None, :], wf3p, bf3p)
    return outp[:B, :1]
